# baseline (speedup 1.0000x reference)
import jax
import jax.numpy as jnp
from jax.experimental import pallas as pl
from jax.experimental.pallas import tpu as pltpu
from jax.experimental.pallas import tpu_sc as plsc

K = 8192
D = 64
TOK = 128
KC = 8192


def _prep_body(w_ref, en2_ref, e2_ref, wp_ref):
    w = w_ref[...]
    wnorm = jnp.sqrt(jnp.sum(w * w, axis=-1, keepdims=True))
    en = w / jnp.maximum(wnorm, 1e-12)
    en2_ref[...] = en + en
    e2_ref[...] = jnp.sum(en * en, axis=-1)[None, :]
    wp_ref[:, :D] = w
    wp_ref[:, D:] = jnp.zeros((K, 128 - D), jnp.float32)


def _tc_prep(W):
    return pl.pallas_call(
        _prep_body,
        in_specs=[pl.BlockSpec((K, D), lambda: (0, 0))],
        out_specs=[
            pl.BlockSpec((K, D), lambda: (0, 0)),
            pl.BlockSpec((1, K), lambda: (0, 0)),
            pl.BlockSpec((K, 128), lambda: (0, 0)),
        ],
        out_shape=[
            jax.ShapeDtypeStruct((K, D), jnp.float32),
            jax.ShapeDtypeStruct((1, K), jnp.float32),
            jax.ShapeDtypeStruct((K, 128), jnp.float32),
        ],
    )(W)


def _dist_argmin_body(x_ref, en2_ref, e2_ref, idx_ref, xn_ref):
    x = x_ref[...]
    xnorm = jnp.sqrt(jnp.sum(x * x, axis=-1, keepdims=True))
    xn = x / jnp.maximum(xnorm, 1e-12)
    xn_ref[...] = xn
    x2 = jnp.sum(xn * xn, axis=-1, keepdims=True)
    best_d = jnp.full((TOK,), jnp.inf, dtype=jnp.float32)
    best_i = jnp.zeros((TOK,), dtype=jnp.int32)
    for c in range(K // KC):
        en2_c = en2_ref[c * KC:(c + 1) * KC, :]
        e2_c = e2_ref[0, c * KC:(c + 1) * KC]
        cross2 = jax.lax.dot_general(
            xn, en2_c, (((1,), (1,)), ((), ())),
            preferred_element_type=jnp.float32)
        d2 = jnp.maximum((x2 + e2_c[None, :]) - cross2, 0.0)
        dist = jnp.sqrt(d2)
        m = jnp.min(dist, axis=-1)
        fi = jnp.argmin(dist, axis=-1).astype(jnp.int32) + c * KC
        if K == KC:
            best_d, best_i = m, fi
        else:
            take = m < best_d
            best_i = jnp.where(take, fi, best_i)
            best_d = jnp.where(take, m, best_d)
    idx_ref[0, 0, :] = best_i


def _tc_dist_argmin(xf, en2, e2):
    n = xf.shape[0]
    g = n // TOK
    return pl.pallas_call(
        _dist_argmin_body,
        grid=(g,),
        in_specs=[
            pl.BlockSpec((TOK, D), lambda i: (i, 0)),
            pl.BlockSpec((K, D), lambda i: (0, 0)),
            pl.BlockSpec((1, K), lambda i: (0, 0)),
        ],
        out_specs=[
            pl.BlockSpec((1, 1, TOK), lambda i: (i, 0, 0)),
            pl.BlockSpec((TOK, D), lambda i: (i, 0)),
        ],
        out_shape=[
            jax.ShapeDtypeStruct((g, 1, TOK), jnp.int32),
            jax.ShapeDtypeStruct((n, D), jnp.float32),
        ],
        compiler_params=pltpu.CompilerParams(
            dimension_semantics=("parallel",)),
    )(xf, en2, e2)


def _sc_gather(Wp, idx_row):
    n = idx_row.shape[1]
    gw = 128
    mesh = plsc.VectorSubcoreMesh(core_axis_name="core",
                                  subcore_axis_name="subcore")

    @pl.kernel(out_type=jax.ShapeDtypeStruct((n, 128), jnp.float32),
               mesh=mesh)
    def kern(w_hbm, i_hbm, o_hbm):
        def body(i_vmem, o_vmem):
            pltpu.sync_copy(w_hbm.at[i_vmem.at[0]], o_vmem)

        pltpu.emit_pipeline(
            body,
            grid=(n // gw,),
            in_specs=[pl.BlockSpec((1, gw), index_map=lambda i: (0, i))],
            out_specs=[pl.BlockSpec((gw, 128), index_map=lambda i: (i, 0))],
            core_axis_name=("core", "subcore"),
            dimension_semantics=(pltpu.PARALLEL,),
        )(i_hbm, o_hbm)

    return kern(Wp, idx_row)


def _st_loss_body(xn_ref, q_ref, qst_ref, loss_ref):
    xn = xn_ref[...]
    q = q_ref[:, :D]
    qst_ref[...] = xn + (q - xn)
    d = q - xn
    m = jnp.mean(d * d)
    loss_ref[0, 0] = m + 0.25 * m


def _tc_st_loss(xn, q):
    n = xn.shape[0]
    return pl.pallas_call(
        _st_loss_body,
        in_specs=[
            pl.BlockSpec((n, D), lambda: (0, 0)),
            pl.BlockSpec((n, 128), lambda: (0, 0)),
        ],
        out_specs=[
            pl.BlockSpec((n, D), lambda: (0, 0)),
            pl.BlockSpec(memory_space=pltpu.SMEM),
        ],
        out_shape=[
            jax.ShapeDtypeStruct((n, D), jnp.float32),
            jax.ShapeDtypeStruct((1, 1), jnp.float32),
        ],
    )(xn, q)


def kernel(x, W):
    b, l, d = x.shape
    xf = x.reshape(b * l, d)
    en2, e2, Wp = _tc_prep(W)
    idx3, xn = _tc_dist_argmin(xf, en2, e2)
    q = _sc_gather(Wp, idx3.reshape(1, b * l))
    qst, loss = _tc_st_loss(xn, q)
    return qst.reshape(b, l, d), idx3.reshape(b, l), loss.reshape(())


# speedup vs baseline: 1.1034x; 1.1034x over previous
import jax
import jax.numpy as jnp
from jax.experimental import pallas as pl
from jax.experimental.pallas import tpu as pltpu
from jax.experimental.pallas import tpu_sc as plsc

K = 8192
D = 64
TOK = 512
KC = 8192


def _prep_body(w_ref, en2_ref, e2_ref, wp_ref):
    w = w_ref[...]
    wnorm = jnp.sqrt(jnp.sum(w * w, axis=-1, keepdims=True))
    en = w / jnp.maximum(wnorm, 1e-12)
    en2_ref[...] = en + en
    e2_ref[...] = jnp.sum(en * en, axis=-1)[None, :]
    wp_ref[:, :D] = w
    wp_ref[:, D:] = jnp.zeros((K, 128 - D), jnp.float32)


def _tc_prep(W):
    return pl.pallas_call(
        _prep_body,
        in_specs=[pl.BlockSpec((K, D), lambda: (0, 0))],
        out_specs=[
            pl.BlockSpec((K, D), lambda: (0, 0)),
            pl.BlockSpec((1, K), lambda: (0, 0)),
            pl.BlockSpec((K, 128), lambda: (0, 0)),
        ],
        out_shape=[
            jax.ShapeDtypeStruct((K, D), jnp.float32),
            jax.ShapeDtypeStruct((1, K), jnp.float32),
            jax.ShapeDtypeStruct((K, 128), jnp.float32),
        ],
    )(W)


def _dist_argmin_body(x_ref, en2_ref, e2_ref, idx_ref, xn_ref):
    x = x_ref[...]
    xnorm = jnp.sqrt(jnp.sum(x * x, axis=-1, keepdims=True))
    xn = x / jnp.maximum(xnorm, 1e-12)
    xn_ref[...] = xn
    x2 = jnp.sum(xn * xn, axis=-1, keepdims=True)
    best_d = jnp.full((TOK,), jnp.inf, dtype=jnp.float32)
    best_i = jnp.zeros((TOK,), dtype=jnp.int32)
    for c in range(K // KC):
        en2_c = en2_ref[c * KC:(c + 1) * KC, :]
        e2_c = e2_ref[0, c * KC:(c + 1) * KC]
        cross2 = jax.lax.dot_general(
            xn, en2_c, (((1,), (1,)), ((), ())),
            preferred_element_type=jnp.float32)
        d2 = jnp.maximum((x2 + e2_c[None, :]) - cross2, 0.0)
        dist = jnp.sqrt(d2)
        m = jnp.min(dist, axis=-1)
        fi = jnp.argmin(dist, axis=-1).astype(jnp.int32) + c * KC
        if K == KC:
            best_d, best_i = m, fi
        else:
            take = m < best_d
            best_i = jnp.where(take, fi, best_i)
            best_d = jnp.where(take, m, best_d)
    idx_ref[0, 0, :] = best_i


def _tc_dist_argmin(xf, en2, e2):
    n = xf.shape[0]
    g = n // TOK
    return pl.pallas_call(
        _dist_argmin_body,
        grid=(g,),
        in_specs=[
            pl.BlockSpec((TOK, D), lambda i: (i, 0)),
            pl.BlockSpec((K, D), lambda i: (0, 0)),
            pl.BlockSpec((1, K), lambda i: (0, 0)),
        ],
        out_specs=[
            pl.BlockSpec((1, 1, TOK), lambda i: (i, 0, 0)),
            pl.BlockSpec((TOK, D), lambda i: (i, 0)),
        ],
        out_shape=[
            jax.ShapeDtypeStruct((g, 1, TOK), jnp.int32),
            jax.ShapeDtypeStruct((n, D), jnp.float32),
        ],
        compiler_params=pltpu.CompilerParams(
            dimension_semantics=("parallel",)),
    )(xf, en2, e2)


def _sc_gather(Wp, idx_row):
    n = idx_row.shape[1]
    gw = 128
    mesh = plsc.VectorSubcoreMesh(core_axis_name="core",
                                  subcore_axis_name="subcore")

    @pl.kernel(out_type=jax.ShapeDtypeStruct((n, 128), jnp.float32),
               mesh=mesh)
    def kern(w_hbm, i_hbm, o_hbm):
        def body(i_vmem, o_vmem):
            pltpu.sync_copy(w_hbm.at[i_vmem.at[0]], o_vmem)

        pltpu.emit_pipeline(
            body,
            grid=(n // gw,),
            in_specs=[pl.BlockSpec((1, gw), index_map=lambda i: (0, i))],
            out_specs=[pl.BlockSpec((gw, 128), index_map=lambda i: (i, 0))],
            core_axis_name=("core", "subcore"),
            dimension_semantics=(pltpu.PARALLEL,),
        )(i_hbm, o_hbm)

    return kern(Wp, idx_row)


def _st_loss_body(xn_ref, q_ref, qst_ref, loss_ref):
    xn = xn_ref[...]
    q = q_ref[:, :D]
    qst_ref[...] = xn + (q - xn)
    d = q - xn
    m = jnp.mean(d * d)
    loss_ref[0, 0] = m + 0.25 * m


def _tc_st_loss(xn, q):
    n = xn.shape[0]
    return pl.pallas_call(
        _st_loss_body,
        in_specs=[
            pl.BlockSpec((n, D), lambda: (0, 0)),
            pl.BlockSpec((n, 128), lambda: (0, 0)),
        ],
        out_specs=[
            pl.BlockSpec((n, D), lambda: (0, 0)),
            pl.BlockSpec(memory_space=pltpu.SMEM),
        ],
        out_shape=[
            jax.ShapeDtypeStruct((n, D), jnp.float32),
            jax.ShapeDtypeStruct((1, 1), jnp.float32),
        ],
    )(xn, q)


def kernel(x, W):
    b, l, d = x.shape
    xf = x.reshape(b * l, d)
    en2, e2, Wp = _tc_prep(W)
    idx3, xn = _tc_dist_argmin(xf, en2, e2)
    q = _sc_gather(Wp, idx3.reshape(1, b * l))
    qst, loss = _tc_st_loss(xn, q)
    return qst.reshape(b, l, d), idx3.reshape(b, l), loss.reshape(())


# speedup vs baseline: 1.1863x; 1.0752x over previous
import jax
import jax.numpy as jnp
from jax.experimental import pallas as pl
from jax.experimental.pallas import tpu as pltpu
from jax.experimental.pallas import tpu_sc as plsc

K = 8192
D = 64
TOK = 1024
KC = 8192


def _prep_body(w_ref, en2_ref, e2_ref, wp_ref):
    w = w_ref[...]
    wnorm = jnp.sqrt(jnp.sum(w * w, axis=-1, keepdims=True))
    en = w / jnp.maximum(wnorm, 1e-12)
    en2_ref[...] = en + en
    e2_ref[...] = jnp.sum(en * en, axis=-1)[None, :]
    wp_ref[:, :D] = w
    wp_ref[:, D:] = jnp.zeros((K, 128 - D), jnp.float32)


def _tc_prep(W):
    return pl.pallas_call(
        _prep_body,
        in_specs=[pl.BlockSpec((K, D), lambda: (0, 0))],
        out_specs=[
            pl.BlockSpec((K, D), lambda: (0, 0)),
            pl.BlockSpec((1, K), lambda: (0, 0)),
            pl.BlockSpec((K, 128), lambda: (0, 0)),
        ],
        out_shape=[
            jax.ShapeDtypeStruct((K, D), jnp.float32),
            jax.ShapeDtypeStruct((1, K), jnp.float32),
            jax.ShapeDtypeStruct((K, 128), jnp.float32),
        ],
    )(W)


def _dist_argmin_body(x_ref, en2_ref, e2_ref, idx_ref, xn_ref):
    x = x_ref[...]
    xnorm = jnp.sqrt(jnp.sum(x * x, axis=-1, keepdims=True))
    xn = x / jnp.maximum(xnorm, 1e-12)
    xn_ref[...] = xn
    x2 = jnp.sum(xn * xn, axis=-1, keepdims=True)
    best_d = jnp.full((TOK,), jnp.inf, dtype=jnp.float32)
    best_i = jnp.zeros((TOK,), dtype=jnp.int32)
    for c in range(K // KC):
        en2_c = en2_ref[c * KC:(c + 1) * KC, :]
        e2_c = e2_ref[0, c * KC:(c + 1) * KC]
        cross2 = jax.lax.dot_general(
            xn, en2_c, (((1,), (1,)), ((), ())),
            preferred_element_type=jnp.float32)
        d2 = jnp.maximum((x2 + e2_c[None, :]) - cross2, 0.0)
        dist = jnp.sqrt(d2)
        m = jnp.min(dist, axis=-1)
        fi = jnp.argmin(dist, axis=-1).astype(jnp.int32) + c * KC
        if K == KC:
            best_d, best_i = m, fi
        else:
            take = m < best_d
            best_i = jnp.where(take, fi, best_i)
            best_d = jnp.where(take, m, best_d)
    idx_ref[0, 0, :] = best_i


def _tc_dist_argmin(xf, en2, e2):
    n = xf.shape[0]
    g = n // TOK
    return pl.pallas_call(
        _dist_argmin_body,
        grid=(g,),
        in_specs=[
            pl.BlockSpec((TOK, D), lambda i: (i, 0)),
            pl.BlockSpec((K, D), lambda i: (0, 0)),
            pl.BlockSpec((1, K), lambda i: (0, 0)),
        ],
        out_specs=[
            pl.BlockSpec((1, 1, TOK), lambda i: (i, 0, 0)),
            pl.BlockSpec((TOK, D), lambda i: (i, 0)),
        ],
        out_shape=[
            jax.ShapeDtypeStruct((g, 1, TOK), jnp.int32),
            jax.ShapeDtypeStruct((n, D), jnp.float32),
        ],
        compiler_params=pltpu.CompilerParams(
            dimension_semantics=("parallel",)),
    )(xf, en2, e2)


def _sc_gather(Wp, idx_row):
    n = idx_row.shape[1]
    gw = 128
    mesh = plsc.VectorSubcoreMesh(core_axis_name="core",
                                  subcore_axis_name="subcore")

    @pl.kernel(out_type=jax.ShapeDtypeStruct((n, 128), jnp.float32),
               mesh=mesh)
    def kern(w_hbm, i_hbm, o_hbm):
        def body(i_vmem, o_vmem):
            pltpu.sync_copy(w_hbm.at[i_vmem.at[0]], o_vmem)

        pltpu.emit_pipeline(
            body,
            grid=(n // gw,),
            in_specs=[pl.BlockSpec((1, gw), index_map=lambda i: (0, i))],
            out_specs=[pl.BlockSpec((gw, 128), index_map=lambda i: (i, 0))],
            core_axis_name=("core", "subcore"),
            dimension_semantics=(pltpu.PARALLEL,),
        )(i_hbm, o_hbm)

    return kern(Wp, idx_row)


def _st_loss_body(xn_ref, q_ref, qst_ref, loss_ref):
    xn = xn_ref[...]
    q = q_ref[:, :D]
    qst_ref[...] = xn + (q - xn)
    d = q - xn
    m = jnp.mean(d * d)
    loss_ref[0, 0] = m + 0.25 * m


def _tc_st_loss(xn, q):
    n = xn.shape[0]
    return pl.pallas_call(
        _st_loss_body,
        in_specs=[
            pl.BlockSpec((n, D), lambda: (0, 0)),
            pl.BlockSpec((n, 128), lambda: (0, 0)),
        ],
        out_specs=[
            pl.BlockSpec((n, D), lambda: (0, 0)),
            pl.BlockSpec(memory_space=pltpu.SMEM),
        ],
        out_shape=[
            jax.ShapeDtypeStruct((n, D), jnp.float32),
            jax.ShapeDtypeStruct((1, 1), jnp.float32),
        ],
    )(xn, q)


def kernel(x, W):
    b, l, d = x.shape
    xf = x.reshape(b * l, d)
    en2, e2, Wp = _tc_prep(W)
    idx3, xn = _tc_dist_argmin(xf, en2, e2)
    q = _sc_gather(Wp, idx3.reshape(1, b * l))
    qst, loss = _tc_st_loss(xn, q)
    return qst.reshape(b, l, d), idx3.reshape(b, l), loss.reshape(())
